# baseline (speedup 1.0000x reference)
import functools

import jax
import jax.numpy as jnp
from jax import lax
from jax.experimental import pallas as pl
from jax.experimental.pallas import tpu as pltpu
from jax.experimental.pallas import tpu_sc as plsc

VOCAB = 100000
EMBED_DIM = 300
PAD_DIM = 304
BATCH = 1024
SEQ = 200

N_IDX = BATCH * SEQ
NUM_WORKERS = 32
PER_WORKER = N_IDX // NUM_WORKERS
CHUNK = 128
NUM_CHUNKS = PER_WORKER // CHUNK
LANES = 16
VREGS_PER_ROW = PAD_DIM // LANES

_mesh = plsc.VectorSubcoreMesh(core_axis_name="c", subcore_axis_name="s")


@functools.partial(
    pl.kernel,
    mesh=_mesh,
    out_type=jax.ShapeDtypeStruct((N_IDX * EMBED_DIM,), jnp.float32),
    scratch_types=[
        pltpu.VMEM((PER_WORKER,), jnp.int32),
        pltpu.VMEM((CHUNK, PAD_DIM), jnp.float32),
        pltpu.VMEM((CHUNK * EMBED_DIM + LANES,), jnp.float32),
        pltpu.SemaphoreType.DMA,
    ],
    compiler_params=pltpu.CompilerParams(use_tc_tiling_on_sc=False),
)
def _gather_kernel(table_hbm, idx_hbm, out_hbm, idx_v, buf, packed, sem):
    wid = lax.axis_index("s") * 2 + lax.axis_index("c")
    base = pl.multiple_of(wid * PER_WORKER, PER_WORKER)
    pltpu.sync_copy(idx_hbm.at[pl.ds(base, PER_WORKER)], idx_v)

    def chunk_body(c, carry):
        off = pl.multiple_of(c * CHUNK, CHUNK)
        idx_chunk = idx_v.at[pl.ds(off, CHUNK)]
        pltpu.async_copy(table_hbm.at[idx_chunk], buf, sem).wait()

        def row_body(i, carry2):
            row = buf.at[i]
            dst = pl.multiple_of(i * EMBED_DIM, 4)
            for k in range(VREGS_PER_ROW):
                packed[pl.ds(dst + k * LANES, LANES)] = row[pl.ds(k * LANES, LANES)]
            return carry2

        lax.fori_loop(0, CHUNK, row_body, 0)

        out_off = pl.multiple_of((base + off) * EMBED_DIM, 8 * EMBED_DIM)
        pltpu.sync_copy(packed.at[pl.ds(0, CHUNK * EMBED_DIM)],
                        out_hbm.at[pl.ds(out_off, CHUNK * EMBED_DIM)])
        return carry

    lax.fori_loop(0, NUM_CHUNKS, chunk_body, 0)


def kernel(word_sequences, embedding):
    table = jnp.pad(embedding, ((0, 0), (0, PAD_DIM - EMBED_DIM)))
    flat_idx = word_sequences.reshape(N_IDX)
    out = _gather_kernel(table, flat_idx)
    return out.reshape(BATCH, SEQ, EMBED_DIM)


# speedup vs baseline: 1.8858x; 1.8858x over previous
import functools

import jax
import jax.numpy as jnp
from jax import lax
from jax.experimental import pallas as pl
from jax.experimental.pallas import tpu as pltpu
from jax.experimental.pallas import tpu_sc as plsc

VOCAB = 100000
EMBED_DIM = 300
PAD_DIM = 384
BATCH = 1024
SEQ = 200

N_IDX = BATCH * SEQ
NUM_WORKERS = 32
PER_WORKER = N_IDX // NUM_WORKERS
CHUNK = 128
NUM_CHUNKS = PER_WORKER // CHUNK

_mesh = plsc.VectorSubcoreMesh(core_axis_name="c", subcore_axis_name="s")


@functools.partial(
    pl.kernel,
    mesh=_mesh,
    out_type=jax.ShapeDtypeStruct((N_IDX, PAD_DIM), jnp.float32),
    scratch_types=[
        pltpu.VMEM((PER_WORKER,), jnp.int32),
        pltpu.VMEM((CHUNK, PAD_DIM), jnp.float32),
        pltpu.SemaphoreType.DMA,
    ],
    compiler_params=pltpu.CompilerParams(use_tc_tiling_on_sc=True),
)
def _gather_kernel(table_hbm, idx_hbm, out_hbm, idx_v, buf, sem):
    wid = lax.axis_index("s") * 2 + lax.axis_index("c")
    base = pl.multiple_of(wid * PER_WORKER, PER_WORKER)
    pltpu.sync_copy(idx_hbm.at[pl.ds(base, PER_WORKER)], idx_v)

    def body(c, carry):
        off = pl.multiple_of(c * CHUNK, CHUNK)
        idx_chunk = idx_v.at[pl.ds(off, CHUNK)]
        pltpu.async_copy(table_hbm.at[idx_chunk], buf, sem).wait()
        out_off = pl.multiple_of(base + off, CHUNK)
        pltpu.sync_copy(buf, out_hbm.at[pl.ds(out_off, CHUNK)])
        return carry

    lax.fori_loop(0, NUM_CHUNKS, body, 0)


def kernel(word_sequences, embedding):
    table = jnp.pad(embedding, ((0, 0), (0, PAD_DIM - EMBED_DIM)))
    flat_idx = word_sequences.reshape(N_IDX)
    out = _gather_kernel(table, flat_idx)
    return out[:, :EMBED_DIM].reshape(BATCH, SEQ, EMBED_DIM)


# speedup vs baseline: 2.5419x; 1.3479x over previous
import functools

import jax
import jax.numpy as jnp
from jax import lax
from jax.experimental import pallas as pl
from jax.experimental.pallas import tpu as pltpu
from jax.experimental.pallas import tpu_sc as plsc

VOCAB = 100000
EMBED_DIM = 300
MAIN_COLS = 256
TAIL_COLS = 128
BATCH = 1024
SEQ = 200

NUM_WORKERS = 32
B_PER_WORKER = BATCH // NUM_WORKERS
CHUNK_A = 104
CHUNK_B = SEQ - CHUNK_A

_mesh = plsc.VectorSubcoreMesh(core_axis_name="c", subcore_axis_name="s")


@functools.partial(
    pl.kernel,
    mesh=_mesh,
    out_type=jax.ShapeDtypeStruct((BATCH, SEQ, EMBED_DIM), jnp.float32),
    scratch_types=[
        pltpu.VMEM((SEQ,), jnp.int32),
        pltpu.VMEM((SEQ, EMBED_DIM), jnp.float32),
        pltpu.VMEM((SEQ, TAIL_COLS), jnp.float32),
        pltpu.SemaphoreType.DMA,
        pltpu.SemaphoreType.DMA,
    ],
    compiler_params=pltpu.CompilerParams(use_tc_tiling_on_sc=True,
                                         needs_layout_passes=False),
)
def _gather_kernel(table_hbm, tail_hbm, idx_hbm, out_hbm, idx_v, buf, tbuf,
                   sem_a, sem_b):
    wid = lax.axis_index("s") * 2 + lax.axis_index("c")
    b0 = pl.multiple_of(wid * B_PER_WORKER, B_PER_WORKER)

    def body(bi, carry):
        b = b0 + bi
        pltpu.sync_copy(idx_hbm.at[b], idx_v)
        idx_a = idx_v.at[pl.ds(0, CHUNK_A)]
        idx_b = idx_v.at[pl.ds(CHUNK_A, CHUNK_B)]
        ca = pltpu.async_copy(
            table_hbm.at[idx_a, pl.ds(0, MAIN_COLS)],
            buf.at[pl.ds(0, CHUNK_A), pl.ds(0, MAIN_COLS)], sem_a)
        cb = pltpu.async_copy(
            table_hbm.at[idx_b, pl.ds(0, MAIN_COLS)],
            buf.at[pl.ds(CHUNK_A, CHUNK_B), pl.ds(0, MAIN_COLS)], sem_a)
        ta = pltpu.async_copy(tail_hbm.at[idx_a],
                              tbuf.at[pl.ds(0, CHUNK_A)], sem_b)
        tb = pltpu.async_copy(tail_hbm.at[idx_b],
                              tbuf.at[pl.ds(CHUNK_A, CHUNK_B)], sem_b)
        ca.wait()
        cb.wait()
        ta.wait()
        tb.wait()

        def row_body(i, carry2):
            buf[i, pl.ds(256, 16)] = tbuf[i, pl.ds(0, 16)]
            buf[i, pl.ds(272, 16)] = tbuf[i, pl.ds(16, 16)]
            tail = tbuf[i, pl.ds(32, 16)]
            rows = jnp.full((16,), i, jnp.int32)
            cols = 288 + lax.iota(jnp.int32, 16)
            mask = lax.iota(jnp.int32, 16) < 12
            plsc.store_scatter(buf, [rows, cols], tail, mask=mask)
            return carry2

        lax.fori_loop(0, SEQ, row_body, 0)

        pltpu.sync_copy(buf, out_hbm.at[b])
        return carry

    lax.fori_loop(0, B_PER_WORKER, body, 0)


def kernel(word_sequences, embedding):
    tail = jnp.pad(embedding[:, MAIN_COLS:],
                   ((0, 0), (0, TAIL_COLS - (EMBED_DIM - MAIN_COLS))))
    return _gather_kernel(embedding, tail, word_sequences)
